# baseline (speedup 1.0000x reference)
import functools

import jax
import jax.numpy as jnp
from jax import lax
from jax.experimental import pallas as pl
from jax.experimental.pallas import tpu as pltpu
from jax.experimental.pallas import tpu_sc as plsc

N, D, F, C, E = 10000, 128, 64, 64, 320000
NC, NS = 2, 16
NW = NC * NS
NP = 10240
B = 80
NBUF = 3
RPT = NP // NS
RB = 256
F3 = 3 * F

NB0, NB1 = 216, 42
NBT = NB0 + NB1
NBMAX = NB0
EROWS = NS * NBT + (NB0 - NB1)
EP = EROWS * B

_mesh = plsc.VectorSubcoreMesh(core_axis_name="c", subcore_axis_name="s")


def _deg_body(cols_h, ws_h, out_h, col_v, w_v, deg_v):
    c = lax.axis_index("c")
    s = lax.axis_index("s")
    wid = s * NC + c
    srow = jnp.where(c == 0, s * NB0, NS * NB0 + s * NB1)
    nb = jnp.where(c == 0, NB0, NB1)
    z = jnp.zeros((16,), jnp.float32)

    def zbody(i, _):
        deg_v[pl.ds(i * 16, 16)] = z
        return 0

    lax.fori_loop(0, 3 * NP // 16, zbody, 0)

    for st in range(3):
        pltpu.sync_copy(cols_h.at[st, pl.ds(srow, NBMAX)], col_v)
        pltpu.sync_copy(ws_h.at[st, pl.ds(srow, NBMAX)], w_v)

        def ebody(i, _, st=st):
            r = i // (B // 16)
            j = i % (B // 16)
            idx = col_v[r, pl.ds(j * 16, 16)] + (st * NP)
            wv = w_v[r, pl.ds(j * 16, 16)]
            plsc.addupdate_scatter(deg_v, [idx], wv)
            return 0

        lax.fori_loop(0, nb * (B // 16), ebody, 0)
    pltpu.sync_copy(deg_v, out_h.at[pl.ds(wid * 3 * NP, 3 * NP)])


_deg = functools.partial(
    pl.kernel,
    out_type=jax.ShapeDtypeStruct((NW * 3 * NP,), jnp.float32),
    mesh=_mesh,
    compiler_params=pltpu.CompilerParams(needs_layout_passes=False, use_tc_tiling_on_sc=False),
    scratch_types=[
        pltpu.VMEM((NBMAX, B), jnp.int32),
        pltpu.VMEM((NBMAX, B), jnp.float32),
        pltpu.VMEM((3 * NP,), jnp.float32),
    ],
)(_deg_body)


def _gs_body(rows_h, cols_h, ws_h, tab_h, out_h,
             idx_r, idx_c, w_v, acc, zb,
             g0, g1, g2, s0, s1, s2,
             gm0, gm1, gm2, sm0, sm1, sm2):
    c = lax.axis_index("c")
    s = lax.axis_index("s")
    wid = s * NC + c
    gbufs = (g0, g1, g2)
    sbufs = (s0, s1, s2)
    gsems = (gm0, gm1, gm2)
    ssems = (sm0, sm1, sm2)

    z = jnp.zeros((16,), jnp.float32)

    def zb_body(i, _):
        for f in range(F // 16):
            zb[i, pl.ds(f * 16, 16)] = z
        return 0

    lax.fori_loop(0, B, zb_body, 0)

    def g_start(st, b, k):
        pltpu.async_copy(tab_h.at[st].at[idx_r.at[b]], gbufs[k], gsems[k])

    def g_wait(st, b, k):
        pltpu.make_async_copy(tab_h.at[st].at[idx_r.at[b]], gbufs[k],
                              gsems[k]).wait()

    def s_start(b, k):
        return

    def s_wait(b, k):
        return

    def scale(b, k):
        return
        gb = gbufs[k]
        sb = sbufs[k]

        def sgrp(j, _):
            wvec = w_v[b, pl.ds(j * 16, 16)]
            base = j * 16
            for e in range(16):
                m = wvec[e]
                r = base + e
                for f in range(F // 16):
                    sb[r, pl.ds(f * 16, 16)] = gb[r, pl.ds(f * 16, 16)] * m
            return 0

        lax.fori_loop(0, B // 16, sgrp, 0)

    srow = jnp.where(c == 0, s * NB0, NS * NB0 + s * NB1)
    nb = jnp.where(c == 0, NB0, NB1)

    for st in range(3):
        for zi in range(RPT // B):
            pltpu.sync_copy(zb, acc.at[pl.ds(s * RPT + zi * B, B)])
        plsc.subcore_barrier()

        pltpu.sync_copy(rows_h.at[st, pl.ds(srow, NBMAX)], idx_r)
        pltpu.sync_copy(cols_h.at[st, pl.ds(srow, NBMAX)], idx_c)
        pltpu.sync_copy(ws_h.at[st, pl.ds(srow, NBMAX)], w_v)

        NB6 = 6
        rbufs = gbufs + sbufs
        rsems = gsems + ssems
        def g_start6(b, k, st=st):
            pltpu.async_copy(tab_h.at[st].at[idx_r.at[b]], rbufs[k], rsems[k])
        def g_wait6(b, k, st=st):
            pltpu.make_async_copy(tab_h.at[st].at[idx_r.at[b]], rbufs[k],
                                  rsems[k]).wait()
        for k in range(NB6):
            g_start6(k, k)
        def mbody(g, _):
            for k in range(NB6):
                b = g * NB6 + k
                g_wait6(b, k)
                g_start6(b + NB6, k)
            return 0
        lax.fori_loop(0, nb // NB6 - 1, mbody, 0)
        for k in range(NB6):
            g_wait6(nb - NB6 + k, k)

        plsc.subcore_barrier()
        pltpu.sync_copy(acc.at[pl.ds(s * RPT, RPT)],
                        out_h.at[c, st, pl.ds(s * RPT, RPT)])
        plsc.subcore_barrier()


_gs = functools.partial(
    pl.kernel,
    out_type=jax.ShapeDtypeStruct((NC, 3, NP, F), jnp.float32),
    mesh=_mesh,
    compiler_params=pltpu.CompilerParams(needs_layout_passes=False, use_tc_tiling_on_sc=False),
    scratch_types=[
        pltpu.VMEM((NBMAX, B), jnp.int32),
        pltpu.VMEM((NBMAX, B), jnp.int32),
        pltpu.VMEM((NBMAX, B), jnp.float32),
        pltpu.VMEM_SHARED((NP, F), jnp.float32),
        pltpu.VMEM((B, F), jnp.float32),
    ]
    + [pltpu.VMEM((B, F), jnp.float32)] * (2 * NBUF)
    + [pltpu.SemaphoreType.DMA] * (2 * NBUF),
)(_gs_body)


def _tc0_body(xp_ref, w1_ref, degp_ref, hp_ref, dis_ref):
    deg = jnp.sum(degp_ref[...], axis=0) + 1.0
    dis = lax.rsqrt(deg)
    dis_ref[...] = dis
    h = jnp.dot(xp_ref[...], w1_ref[...], preferred_element_type=jnp.float32)
    hp_ref[...] = dis[:, :, None] * h[None, :, :]


_tc0 = pl.pallas_call(
    _tc0_body,
    grid=(NP // RB,),
    in_specs=[
        pl.BlockSpec((RB, D), lambda i: (i, 0)),
        pl.BlockSpec((D, F), lambda i: (0, 0)),
        pl.BlockSpec((NW, 3, RB), lambda i: (0, 0, i)),
    ],
    out_specs=[
        pl.BlockSpec((3, RB, F), lambda i: (0, i, 0)),
        pl.BlockSpec((3, RB), lambda i: (0, i)),
    ],
    out_shape=[
        jax.ShapeDtypeStruct((3, NP, F), jnp.float32),
        jax.ShapeDtypeStruct((3, NP), jnp.float32),
    ],
)


def _combine(acc_ref, hp_ref, dis_ref, b_ref):
    accs = acc_ref[...]
    dis = dis_ref[...]
    h = dis[:, :, None] * (accs[0] + accs[1] + hp_ref[...]) + b_ref[...]
    x = jnp.maximum(h, 0.0)
    return jnp.concatenate([x[0], x[1], x[2]], axis=-1), dis


def _tc1_body(acc_ref, hp_ref, dis_ref, w2_ref, b1_ref, out_ref):
    xcat, dis = _combine(acc_ref, hp_ref, dis_ref, b1_ref)
    h2 = jnp.dot(xcat, w2_ref[...], preferred_element_type=jnp.float32)
    out_ref[...] = dis[:, :, None] * h2[None, :, :]


_tc1 = pl.pallas_call(
    _tc1_body,
    grid=(NP // RB,),
    in_specs=[
        pl.BlockSpec((NC, 3, RB, F), lambda i: (0, 0, i, 0)),
        pl.BlockSpec((3, RB, F), lambda i: (0, i, 0)),
        pl.BlockSpec((3, RB), lambda i: (0, i)),
        pl.BlockSpec((F3, F), lambda i: (0, 0)),
        pl.BlockSpec((1, F), lambda i: (0, 0)),
    ],
    out_specs=pl.BlockSpec((3, RB, F), lambda i: (0, i, 0)),
    out_shape=jax.ShapeDtypeStruct((3, NP, F), jnp.float32),
)


def _tc2_body(acc_ref, hp_ref, dis_ref, cw_ref, cb_ref, b2_ref, out_ref):
    xcat, _ = _combine(acc_ref, hp_ref, dis_ref, b2_ref)
    logits = jnp.dot(xcat, cw_ref[...], preferred_element_type=jnp.float32)
    logits = logits + cb_ref[...][None, :]
    m = jnp.max(logits, axis=1, keepdims=True)
    lse = jnp.log(jnp.sum(jnp.exp(logits - m), axis=1, keepdims=True)) + m
    out_ref[...] = logits - lse


_tc2 = pl.pallas_call(
    _tc2_body,
    grid=(NP // RB,),
    in_specs=[
        pl.BlockSpec((NC, 3, RB, F), lambda i: (0, 0, i, 0)),
        pl.BlockSpec((3, RB, F), lambda i: (0, i, 0)),
        pl.BlockSpec((3, RB), lambda i: (0, i)),
        pl.BlockSpec((F3, C), lambda i: (0, 0)),
        pl.BlockSpec((C,), lambda i: (0,)),
        pl.BlockSpec((1, F), lambda i: (0, 0)),
    ],
    out_specs=pl.BlockSpec((RB, C), lambda i: (i, 0)),
    out_shape=jax.ShapeDtypeStruct((NP, C), jnp.float32),
)


def kernel(x, edge_index, edge_in, edge_out, in_w, out_w,
           lin1_w, lin2_w, conv_w, conv_b, bias1, bias2):
    xp = jnp.pad(x, ((0, NP - N), (0, 0)))
    pad_i = jnp.full((EP - E,), NP - 1, jnp.int32)
    pad_w = jnp.zeros((EP - E,), jnp.float32)
    ones_e = jnp.ones((E,), jnp.float32)

    rows, cols, ws = [], [], []
    for ei, w in ((edge_index, ones_e), (edge_in, in_w), (edge_out, out_w)):
        rows.append(jnp.concatenate([ei[0], pad_i]))
        cols.append(jnp.concatenate([ei[1], pad_i]))
        ws.append(jnp.concatenate([w, pad_w]))
    rows3 = jnp.stack(rows).reshape(3, EROWS, B)
    cols3 = jnp.stack(cols).reshape(3, EROWS, B)
    ws3 = jnp.stack(ws).reshape(3, EROWS, B)

    degp = _deg(cols3, ws3).reshape(NW, 3, NP)
    hp, dis = _tc0(xp, lin1_w, degp)
    acc1 = _gs(rows3, cols3, ws3, hp)
    hp2 = _tc1(acc1, hp, dis, lin2_w, bias1)
    acc2 = _gs(rows3, cols3, ws3, hp2)
    out = _tc2(acc2, hp2, dis, conv_w, conv_b, bias2)
    return out[:N]


# speedup vs baseline: 1.0095x; 1.0095x over previous
import functools

import jax
import jax.numpy as jnp
from jax import lax
from jax.experimental import pallas as pl
from jax.experimental.pallas import tpu as pltpu
from jax.experimental.pallas import tpu_sc as plsc

N, D, F, C, E = 10000, 128, 64, 64, 320000
NC, NS = 2, 16
NW = NC * NS
NP = 10240
B = 80
NBUF = 3
RPT = NP // NS
RB = 256
F3 = 3 * F

NB0, NB1 = 240, 18
NBT = NB0 + NB1
NBMAX = NB0
EROWS = NS * NBT + (NB0 - NB1)
EP = EROWS * B

_mesh = plsc.VectorSubcoreMesh(core_axis_name="c", subcore_axis_name="s")


def _deg_body(cols_h, ws_h, out_h, col_v, w_v, deg_v):
    c = lax.axis_index("c")
    s = lax.axis_index("s")
    wid = s * NC + c
    srow = jnp.where(c == 0, s * NB0, NS * NB0 + s * NB1)
    nb = jnp.where(c == 0, NB0, NB1)
    z = jnp.zeros((16,), jnp.float32)

    def zbody(i, _):
        deg_v[pl.ds(i * 16, 16)] = z
        return 0

    lax.fori_loop(0, 3 * NP // 16, zbody, 0)

    for st in range(3):
        pltpu.sync_copy(cols_h.at[st, pl.ds(srow, NBMAX)], col_v)
        pltpu.sync_copy(ws_h.at[st, pl.ds(srow, NBMAX)], w_v)

        def ebody(i, _, st=st):
            r = i // (B // 16)
            j = i % (B // 16)
            idx = col_v[r, pl.ds(j * 16, 16)] + (st * NP)
            wv = w_v[r, pl.ds(j * 16, 16)]
            plsc.addupdate_scatter(deg_v, [idx], wv)
            return 0

        lax.fori_loop(0, nb * (B // 16), ebody, 0)
    pltpu.sync_copy(deg_v, out_h.at[pl.ds(wid * 3 * NP, 3 * NP)])


_deg = functools.partial(
    pl.kernel,
    out_type=jax.ShapeDtypeStruct((NW * 3 * NP,), jnp.float32),
    mesh=_mesh,
    compiler_params=pltpu.CompilerParams(needs_layout_passes=False, use_tc_tiling_on_sc=False),
    scratch_types=[
        pltpu.VMEM((NBMAX, B), jnp.int32),
        pltpu.VMEM((NBMAX, B), jnp.float32),
        pltpu.VMEM((3 * NP,), jnp.float32),
    ],
)(_deg_body)


def _gs_body(rows_h, cols_h, ws_h, tab_h, out_h,
             idx_r, idx_c, w_v, acc, zb,
             g0, g1, g2, s0, s1, s2,
             gm0, gm1, gm2, sm0, sm1, sm2):
    c = lax.axis_index("c")
    s = lax.axis_index("s")
    gbufs = (g0, g1, g2)
    sbufs = (s0, s1, s2)
    gsems = (gm0, gm1, gm2)
    ssems = (sm0, sm1, sm2)

    z = jnp.zeros((16,), jnp.float32)

    def zb_body(i, _):
        for f in range(F // 16):
            zb[i, pl.ds(f * 16, 16)] = z
        return 0

    lax.fori_loop(0, 16, zb_body, 0)

    def g_start(st, b, k):
        pltpu.async_copy(tab_h.at[st].at[idx_r.at[b]], gbufs[k], gsems[k])

    def g_wait(st, b, k):
        pltpu.make_async_copy(tab_h.at[st].at[idx_r.at[b]], gbufs[k],
                              gsems[k]).wait()

    def s_start(b, k, frm):
        pltpu.async_copy(frm[k], acc.at[idx_c.at[b]], ssems[k], add=True)

    def s_wait(b, k, frm):
        pltpu.make_async_copy(frm[k], acc.at[idx_c.at[b]], ssems[k]).wait()

    def scale(b, k):
        gb = gbufs[k]
        sb = sbufs[k]

        def sgrp(j, _):
            wvec = w_v[b, pl.ds(j * 16, 16)]
            base = j * 16
            for e in range(16):
                m = wvec[e]
                r = base + e
                for f in range(F // 16):
                    sb[r, pl.ds(f * 16, 16)] = gb[r, pl.ds(f * 16, 16)] * m
            return 0

        lax.fori_loop(0, B // 16, sgrp, 0)

    srow = jnp.where(c == 0, s * NB0, NS * NB0 + s * NB1)
    nb = jnp.where(c == 0, NB0, NB1)

    for st in range(3):
        for zi in range(RPT // 16):
            pltpu.sync_copy(zb, acc.at[pl.ds(s * RPT + zi * 16, 16)])
        plsc.subcore_barrier()

        pltpu.sync_copy(rows_h.at[st, pl.ds(srow, NBMAX)], idx_r)
        pltpu.sync_copy(cols_h.at[st, pl.ds(srow, NBMAX)], idx_c)
        if st > 0:
            pltpu.sync_copy(ws_h.at[st, pl.ds(srow, NBMAX)], w_v)

        for k in range(NBUF):
            g_start(st, k, k)

        if st == 0:
            def mbody0(g, _, st=st):
                for k in range(NBUF):
                    b = g * NBUF + k
                    g_wait(st, b, k)
                    s_start(b, k, gbufs)
                    s_wait(b, k, gbufs)
                    g_start(st, b + NBUF, k)
                return 0

            lax.fori_loop(0, nb // NBUF - 1, mbody0, 0)
            for k in range(NBUF):
                b = nb - NBUF + k
                g_wait(st, b, k)
                s_start(b, k, gbufs)
                s_wait(b, k, gbufs)
        else:
            for k in range(NBUF):
                g_wait(st, k, k)
                scale(k, k)
                s_start(k, k, sbufs)
                g_start(st, k + NBUF, k)

            def mbody(g, _, st=st):
                for k in range(NBUF):
                    b = g * NBUF + k
                    g_wait(st, b, k)
                    s_wait(b - NBUF, k, sbufs)
                    scale(b, k)
                    s_start(b, k, sbufs)
                    g_start(st, b + NBUF, k)
                return 0

            lax.fori_loop(1, nb // NBUF - 1, mbody, 0)

            for k in range(NBUF):
                b = nb - NBUF + k
                g_wait(st, b, k)
                s_wait(b - NBUF, k, sbufs)
                scale(b, k)
                s_start(b, k, sbufs)
            for k in range(NBUF):
                s_wait(nb - NBUF + k, k, sbufs)

        plsc.subcore_barrier()
        pltpu.sync_copy(acc.at[pl.ds(s * RPT, RPT)],
                        out_h.at[c, st, pl.ds(s * RPT, RPT)])
        plsc.subcore_barrier()


_gs = functools.partial(
    pl.kernel,
    out_type=jax.ShapeDtypeStruct((NC, 3, NP, F), jnp.float32),
    mesh=_mesh,
    compiler_params=pltpu.CompilerParams(needs_layout_passes=False, use_tc_tiling_on_sc=False),
    scratch_types=[
        pltpu.VMEM((NBMAX, B), jnp.int32),
        pltpu.VMEM((NBMAX, B), jnp.int32),
        pltpu.VMEM((NBMAX, B), jnp.float32),
        pltpu.VMEM_SHARED((NP, F), jnp.float32),
        pltpu.VMEM((16, F), jnp.float32),
    ]
    + [pltpu.VMEM((B, F), jnp.float32)] * (2 * NBUF)
    + [pltpu.SemaphoreType.DMA] * (2 * NBUF),
)(_gs_body)


def _tc0_body(xp_ref, w1_ref, degp_ref, hp_ref, dis_ref):
    deg = jnp.sum(degp_ref[...], axis=0) + 1.0
    dis = lax.rsqrt(deg)
    dis_ref[...] = dis
    h = jnp.dot(xp_ref[...], w1_ref[...], preferred_element_type=jnp.float32)
    hp_ref[...] = dis[:, :, None] * h[None, :, :]


_tc0 = pl.pallas_call(
    _tc0_body,
    grid=(NP // RB,),
    in_specs=[
        pl.BlockSpec((RB, D), lambda i: (i, 0)),
        pl.BlockSpec((D, F), lambda i: (0, 0)),
        pl.BlockSpec((NW, 3, RB), lambda i: (0, 0, i)),
    ],
    out_specs=[
        pl.BlockSpec((3, RB, F), lambda i: (0, i, 0)),
        pl.BlockSpec((3, RB), lambda i: (0, i)),
    ],
    out_shape=[
        jax.ShapeDtypeStruct((3, NP, F), jnp.float32),
        jax.ShapeDtypeStruct((3, NP), jnp.float32),
    ],
)


def _combine(acc_ref, hp_ref, dis_ref, b_ref):
    accs = acc_ref[...]
    dis = dis_ref[...]
    h = dis[:, :, None] * (accs[0] + accs[1] + hp_ref[...]) + b_ref[...]
    x = jnp.maximum(h, 0.0)
    return jnp.concatenate([x[0], x[1], x[2]], axis=-1), dis


def _tc1_body(acc_ref, hp_ref, dis_ref, w2_ref, b1_ref, out_ref):
    xcat, dis = _combine(acc_ref, hp_ref, dis_ref, b1_ref)
    h2 = jnp.dot(xcat, w2_ref[...], preferred_element_type=jnp.float32)
    out_ref[...] = dis[:, :, None] * h2[None, :, :]


_tc1 = pl.pallas_call(
    _tc1_body,
    grid=(NP // RB,),
    in_specs=[
        pl.BlockSpec((NC, 3, RB, F), lambda i: (0, 0, i, 0)),
        pl.BlockSpec((3, RB, F), lambda i: (0, i, 0)),
        pl.BlockSpec((3, RB), lambda i: (0, i)),
        pl.BlockSpec((F3, F), lambda i: (0, 0)),
        pl.BlockSpec((1, F), lambda i: (0, 0)),
    ],
    out_specs=pl.BlockSpec((3, RB, F), lambda i: (0, i, 0)),
    out_shape=jax.ShapeDtypeStruct((3, NP, F), jnp.float32),
)


def _tc2_body(acc_ref, hp_ref, dis_ref, cw_ref, cb_ref, b2_ref, out_ref):
    xcat, _ = _combine(acc_ref, hp_ref, dis_ref, b2_ref)
    logits = jnp.dot(xcat, cw_ref[...], preferred_element_type=jnp.float32)
    logits = logits + cb_ref[...][None, :]
    m = jnp.max(logits, axis=1, keepdims=True)
    lse = jnp.log(jnp.sum(jnp.exp(logits - m), axis=1, keepdims=True)) + m
    out_ref[...] = logits - lse


_tc2 = pl.pallas_call(
    _tc2_body,
    grid=(NP // RB,),
    in_specs=[
        pl.BlockSpec((NC, 3, RB, F), lambda i: (0, 0, i, 0)),
        pl.BlockSpec((3, RB, F), lambda i: (0, i, 0)),
        pl.BlockSpec((3, RB), lambda i: (0, i)),
        pl.BlockSpec((F3, C), lambda i: (0, 0)),
        pl.BlockSpec((C,), lambda i: (0,)),
        pl.BlockSpec((1, F), lambda i: (0, 0)),
    ],
    out_specs=pl.BlockSpec((RB, C), lambda i: (i, 0)),
    out_shape=jax.ShapeDtypeStruct((NP, C), jnp.float32),
)


def kernel(x, edge_index, edge_in, edge_out, in_w, out_w,
           lin1_w, lin2_w, conv_w, conv_b, bias1, bias2):
    xp = jnp.pad(x, ((0, NP - N), (0, 0)))
    pad_i = jnp.full((EP - E,), NP - 1, jnp.int32)
    pad_w = jnp.zeros((EP - E,), jnp.float32)
    ones_e = jnp.ones((E,), jnp.float32)

    rows, cols, ws = [], [], []
    for ei, w in ((edge_index, ones_e), (edge_in, in_w), (edge_out, out_w)):
        rows.append(jnp.concatenate([ei[0], pad_i]))
        cols.append(jnp.concatenate([ei[1], pad_i]))
        ws.append(jnp.concatenate([w, pad_w]))
    rows3 = jnp.stack(rows).reshape(3, EROWS, B)
    cols3 = jnp.stack(cols).reshape(3, EROWS, B)
    ws3 = jnp.stack(ws).reshape(3, EROWS, B)

    degp = _deg(cols3, ws3).reshape(NW, 3, NP)
    hp, dis = _tc0(xp, lin1_w, degp)
    acc1 = _gs(rows3, cols3, ws3, hp)
    hp2 = _tc1(acc1, hp, dis, lin2_w, bias1)
    acc2 = _gs(rows3, cols3, ws3, hp2)
    out = _tc2(acc2, hp2, dis, conv_w, conv_b, bias2)
    return out[:N]


# speedup vs baseline: 2.5607x; 2.5368x over previous
import functools

import jax
import jax.numpy as jnp
from jax import lax
from jax.experimental import pallas as pl
from jax.experimental.pallas import tpu as pltpu
from jax.experimental.pallas import tpu_sc as plsc

N, D, F, C, E = 10000, 128, 64, 64, 320000
NC, NS = 2, 16
NW = NC * NS
NP = 10240
B = 80
NBUF = 4
RPT = NP // NS
RB = 256
F3 = 3 * F

EB = E // B
CHUNK = 128
TPB = 2 * CHUNK

_mesh = plsc.VectorSubcoreMesh(core_axis_name="c", subcore_axis_name="s")


def _deg_body(cola_h, colb_h, colc_h, wb_h, wc_h, out_h, col_v, w_v, deg_v):
    c = lax.axis_index("c")
    s = lax.axis_index("s")
    wid = s * NC + c
    srow = wid * (EB // NW)
    nbd = EB // NW
    z = jnp.zeros((16,), jnp.float32)
    one = jnp.ones((16,), jnp.float32)

    def zbody(i, _):
        deg_v[pl.ds(i * 16, 16)] = z
        return 0

    lax.fori_loop(0, 3 * NP // 16, zbody, 0)

    for st, (col_h, w_h) in enumerate(
            ((cola_h, None), (colb_h, wb_h), (colc_h, wc_h))):
        pltpu.sync_copy(col_h.at[pl.ds(srow, EB // NW)], col_v)
        if w_h is not None:
            pltpu.sync_copy(w_h.at[pl.ds(srow, EB // NW)], w_v)

        def ebody(i, _, st=st, has_w=w_h is not None):
            r = i // (B // 16)
            j = i % (B // 16)
            idx = col_v[r, pl.ds(j * 16, 16)] + (st * NP)
            wv = w_v[r, pl.ds(j * 16, 16)] if has_w else one
            plsc.addupdate_scatter(deg_v, [idx], wv)
            return 0

        lax.fori_loop(0, nbd * (B // 16), ebody, 0)
    pltpu.sync_copy(deg_v, out_h.at[pl.ds(wid * 3 * NP, 3 * NP)])


_deg = functools.partial(
    pl.kernel,
    out_type=jax.ShapeDtypeStruct((NW * 3 * NP,), jnp.float32),
    mesh=_mesh,
    compiler_params=pltpu.CompilerParams(needs_layout_passes=False, use_tc_tiling_on_sc=False),
    scratch_types=[
        pltpu.VMEM((EB // NW, B), jnp.int32),
        pltpu.VMEM((EB // NW, B), jnp.float32),
        pltpu.VMEM((3 * NP,), jnp.float32),
    ],
)(_deg_body)


def _gs_body(rowa_h, rowb_h, rowc_h, cola_h, colb_h, colc_h, wb_h, wc_h,
             tab_h, out_h,
             idx_r, idx_c, w_v, acc, zb,
             g0, g1, g2, g3, s0, s1, s2, s3,
             gm0, gm1, gm2, gm3, sm0, sm1, sm2, sm3):
    c = lax.axis_index("c")
    s = lax.axis_index("s")
    gbufs = (g0, g1, g2, g3)
    sbufs = (s0, s1, s2, s3)
    gsems = (gm0, gm1, gm2, gm3)
    ssems = (sm0, sm1, sm2, sm3)

    z = jnp.zeros((16,), jnp.float32)

    def zb_body(i, _):
        for f in range(F // 16):
            zb[i, pl.ds(f * 16, 16)] = z
        return 0

    lax.fori_loop(0, 16, zb_body, 0)

    def process_set(st, tab_h, row_h, col_h, w_h):
        has_w = w_h is not None

        def g_start(b, k):
            pltpu.async_copy(tab_h.at[idx_r.at[b]], gbufs[k], gsems[k])

        def g_wait(b, k):
            pltpu.make_async_copy(tab_h.at[idx_r.at[b]], gbufs[k],
                                  gsems[k]).wait()

        def s_start(b, k):
            pltpu.async_copy(sbufs[k], acc.at[idx_c.at[b]], ssems[k], add=True)

        def s_wait(b, k):
            pltpu.make_async_copy(sbufs[k], acc.at[idx_c.at[b]],
                                  ssems[k]).wait()

        def scale(b, k):
            gb = gbufs[k]
            sb = sbufs[k]

            def sgrp(j, _):
                base = j * 16
                if has_w:
                    wvec = w_v[b, pl.ds(j * 16, 16)]
                for e in range(16):
                    r = base + e
                    for f in range(F // 16):
                        v = gb[r, pl.ds(f * 16, 16)]
                        if has_w:
                            v = v * wvec[e]
                        sb[r, pl.ds(f * 16, 16)] = v
                return 0

            lax.fori_loop(0, B // 16, sgrp, 0)

        for zi in range(RPT // 16):
            pltpu.sync_copy(zb, acc.at[pl.ds(s * RPT + zi * 16, 16)])
        plsc.subcore_barrier()

        def chunk_body(ch, _):
            base = s * TPB + ch * CHUNK
            srow = jnp.minimum(base, EB - CHUNK)
            bo = base - srow
            nb = jnp.minimum(CHUNK, EB - base)

            pltpu.sync_copy(row_h.at[pl.ds(srow, CHUNK)], idx_r)
            pltpu.sync_copy(col_h.at[pl.ds(srow, CHUNK)], idx_c)
            if has_w:
                pltpu.sync_copy(w_h.at[pl.ds(srow, CHUNK)], w_v)

            for k in range(NBUF):
                g_start(bo + k, k)

            def mbody(g, _):
                for k in range(NBUF):
                    b = bo + g * NBUF + k
                    g_wait(b, k)

                    @pl.when(g > 0)
                    def _():
                        s_wait(b - NBUF, k)

                    scale(b, k)
                    s_start(b, k)

                    @pl.when(g * NBUF + k + NBUF < nb)
                    def _():
                        g_start(b + NBUF, k)
                return 0

            lax.fori_loop(0, nb // NBUF, mbody, 0)
            for k in range(NBUF):
                s_wait(bo + nb - NBUF + k, k)
            return 0

        lax.fori_loop(0, 2, chunk_body, 0)

        plsc.subcore_barrier()
        pltpu.sync_copy(acc.at[pl.ds(s * RPT, RPT)],
                        out_h.at[st, pl.ds(s * RPT, RPT)])
        plsc.subcore_barrier()

    @pl.when(c == 1)
    def _():
        process_set(0, tab_h.at[0], rowa_h, cola_h, None)

    @pl.when(c == 0)
    def _():
        process_set(1, tab_h.at[1], rowb_h, colb_h, wb_h)
        process_set(2, tab_h.at[2], rowc_h, colc_h, wc_h)


_gs = functools.partial(
    pl.kernel,
    out_type=jax.ShapeDtypeStruct((3, NP, F), jnp.float32),
    mesh=_mesh,
    compiler_params=pltpu.CompilerParams(needs_layout_passes=False, use_tc_tiling_on_sc=False),
    scratch_types=[
        pltpu.VMEM((CHUNK, B), jnp.int32),
        pltpu.VMEM((CHUNK, B), jnp.int32),
        pltpu.VMEM((CHUNK, B), jnp.float32),
        pltpu.VMEM_SHARED((NP, F), jnp.float32),
        pltpu.VMEM((16, F), jnp.float32),
    ]
    + [pltpu.VMEM((B, F), jnp.float32)] * (2 * NBUF)
    + [pltpu.SemaphoreType.DMA] * (2 * NBUF),
)(_gs_body)


def _tc0_body(xp_ref, w1_ref, degp_ref, hp_ref, dis_ref):
    deg = jnp.sum(degp_ref[...], axis=0) + 1.0
    dis = lax.rsqrt(deg)
    dis_ref[...] = dis
    h = jnp.dot(xp_ref[...], w1_ref[...], preferred_element_type=jnp.float32)
    hp_ref[...] = dis[:, :, None] * h[None, :, :]


_tc0 = pl.pallas_call(
    _tc0_body,
    grid=(NP // RB,),
    in_specs=[
        pl.BlockSpec((RB, D), lambda i: (i, 0)),
        pl.BlockSpec((D, F), lambda i: (0, 0)),
        pl.BlockSpec((NW, 3, RB), lambda i: (0, 0, i)),
    ],
    out_specs=[
        pl.BlockSpec((3, RB, F), lambda i: (0, i, 0)),
        pl.BlockSpec((3, RB), lambda i: (0, i)),
    ],
    out_shape=[
        jax.ShapeDtypeStruct((3, NP, F), jnp.float32),
        jax.ShapeDtypeStruct((3, NP), jnp.float32),
    ],
)


def _combine(acc_ref, hp_ref, dis_ref, b_ref):
    dis = dis_ref[...]
    h = dis[:, :, None] * (acc_ref[...] + hp_ref[...]) + b_ref[...]
    x = jnp.maximum(h, 0.0)
    return jnp.concatenate([x[0], x[1], x[2]], axis=-1), dis


def _tc1_body(acc_ref, hp_ref, dis_ref, w2_ref, b1_ref, out_ref):
    xcat, dis = _combine(acc_ref, hp_ref, dis_ref, b1_ref)
    h2 = jnp.dot(xcat, w2_ref[...], preferred_element_type=jnp.float32)
    out_ref[...] = dis[:, :, None] * h2[None, :, :]


_tc1 = pl.pallas_call(
    _tc1_body,
    grid=(NP // RB,),
    in_specs=[
        pl.BlockSpec((3, RB, F), lambda i: (0, i, 0)),
        pl.BlockSpec((3, RB, F), lambda i: (0, i, 0)),
        pl.BlockSpec((3, RB), lambda i: (0, i)),
        pl.BlockSpec((F3, F), lambda i: (0, 0)),
        pl.BlockSpec((1, F), lambda i: (0, 0)),
    ],
    out_specs=pl.BlockSpec((3, RB, F), lambda i: (0, i, 0)),
    out_shape=jax.ShapeDtypeStruct((3, NP, F), jnp.float32),
)


def _tc2_body(acc_ref, hp_ref, dis_ref, cw_ref, cb_ref, b2_ref, out_ref):
    xcat, _ = _combine(acc_ref, hp_ref, dis_ref, b2_ref)
    logits = jnp.dot(xcat, cw_ref[...], preferred_element_type=jnp.float32)
    logits = logits + cb_ref[...][None, :]
    m = jnp.max(logits, axis=1, keepdims=True)
    lse = jnp.log(jnp.sum(jnp.exp(logits - m), axis=1, keepdims=True)) + m
    out_ref[...] = logits - lse


_tc2 = pl.pallas_call(
    _tc2_body,
    grid=(NP // RB,),
    in_specs=[
        pl.BlockSpec((3, RB, F), lambda i: (0, i, 0)),
        pl.BlockSpec((3, RB, F), lambda i: (0, i, 0)),
        pl.BlockSpec((3, RB), lambda i: (0, i)),
        pl.BlockSpec((F3, C), lambda i: (0, 0)),
        pl.BlockSpec((C,), lambda i: (0,)),
        pl.BlockSpec((1, F), lambda i: (0, 0)),
    ],
    out_specs=pl.BlockSpec((RB, C), lambda i: (i, 0)),
    out_shape=jax.ShapeDtypeStruct((NP, C), jnp.float32),
)


def kernel(x, edge_index, edge_in, edge_out, in_w, out_w,
           lin1_w, lin2_w, conv_w, conv_b, bias1, bias2):
    xp = jnp.pad(x, ((0, NP - N), (0, 0)))
    ra = edge_index[0].reshape(EB, B)
    ca = edge_index[1].reshape(EB, B)
    rb = edge_in[0].reshape(EB, B)
    cb = edge_in[1].reshape(EB, B)
    rc = edge_out[0].reshape(EB, B)
    cc = edge_out[1].reshape(EB, B)
    wb = in_w.reshape(EB, B)
    wc = out_w.reshape(EB, B)

    degp = _deg(ca, cb, cc, wb, wc).reshape(NW, 3, NP)
    hp, dis = _tc0(xp, lin1_w, degp)
    acc1 = _gs(ra, rb, rc, ca, cb, cc, wb, wc, hp)
    hp2 = _tc1(acc1, hp, dis, lin2_w, bias1)
    acc2 = _gs(ra, rb, rc, ca, cb, cc, wb, wc, hp2)
    out = _tc2(acc2, hp2, dis, conv_w, conv_b, bias2)
    return out[:N]


# speedup vs baseline: 2.8174x; 1.1002x over previous
import functools

import jax
import jax.numpy as jnp
from jax import lax
from jax.experimental import pallas as pl
from jax.experimental.pallas import tpu as pltpu
from jax.experimental.pallas import tpu_sc as plsc

N, D, F, C, E = 10000, 128, 64, 64, 320000
NC, NS = 2, 16
NW = NC * NS
NP = 10240
B = 80
NBUF = 4
RPT = NP // NS
RB = 256
F3 = 3 * F

EB = E // B
CHUNK = 128
TPB = 2 * CHUNK
BMT = 152
BXT = 100

_mesh = plsc.VectorSubcoreMesh(core_axis_name="c", subcore_axis_name="s")


def _deg_body(cola_h, colb_h, colc_h, wb_h, wc_h, out_h, col_v, w_v, deg_v):
    c = lax.axis_index("c")
    s = lax.axis_index("s")
    wid = s * NC + c
    srow = wid * (EB // NW)
    nbd = EB // NW
    z = jnp.zeros((16,), jnp.float32)
    one = jnp.ones((16,), jnp.float32)

    def zbody(i, _):
        deg_v[pl.ds(i * 16, 16)] = z
        return 0

    lax.fori_loop(0, 3 * NP // 16, zbody, 0)

    for st, (col_h, w_h) in enumerate(
            ((cola_h, None), (colb_h, wb_h), (colc_h, wc_h))):
        pltpu.sync_copy(col_h.at[pl.ds(srow, EB // NW)], col_v)
        if w_h is not None:
            pltpu.sync_copy(w_h.at[pl.ds(srow, EB // NW)], w_v)

        def ebody(i, _, st=st, has_w=w_h is not None):
            r = i // (B // 16)
            j = i % (B // 16)
            idx = col_v[r, pl.ds(j * 16, 16)] + (st * NP)
            wv = w_v[r, pl.ds(j * 16, 16)] if has_w else one
            plsc.addupdate_scatter(deg_v, [idx], wv)
            return 0

        lax.fori_loop(0, nbd * (B // 16), ebody, 0)
    pltpu.sync_copy(deg_v, out_h.at[pl.ds(wid * 3 * NP, 3 * NP)])


_deg = functools.partial(
    pl.kernel,
    out_type=jax.ShapeDtypeStruct((NW * 3 * NP,), jnp.float32),
    mesh=_mesh,
    compiler_params=pltpu.CompilerParams(needs_layout_passes=False, use_tc_tiling_on_sc=False),
    scratch_types=[
        pltpu.VMEM((EB // NW, B), jnp.int32),
        pltpu.VMEM((EB // NW, B), jnp.float32),
        pltpu.VMEM((3 * NP,), jnp.float32),
    ],
)(_deg_body)


def _gs_body(rowa_h, rowb_h, rowc_h, cola_h, colb_h, colc_h, wb_h, wc_h,
             tab_h, out_h,
             idx_r, idx_c, w_v, acc, zb,
             g0, g1, g2, g3, s0, s1, s2, s3,
             gm0, gm1, gm2, gm3, sm0, sm1, sm2, sm3):
    c = lax.axis_index("c")
    s = lax.axis_index("s")
    gbufs = (g0, g1, g2, g3)
    sbufs = (s0, s1, s2, s3)
    gsems = (gm0, gm1, gm2, gm3)
    ssems = (sm0, sm1, sm2, sm3)

    z = jnp.zeros((16,), jnp.float32)

    def zb_body(i, _):
        for f in range(F // 16):
            zb[i, pl.ds(f * 16, 16)] = z
        return 0

    lax.fori_loop(0, 16, zb_body, 0)

    def process_set(st, tab_h, row_h, col_h, w_h, start=0, pt=TPB,
                    nchunks=2):
        has_w = w_h is not None

        def g_start(b, k):
            pltpu.async_copy(tab_h.at[idx_r.at[b]], gbufs[k], gsems[k])

        def g_wait(b, k):
            pltpu.make_async_copy(tab_h.at[idx_r.at[b]], gbufs[k],
                                  gsems[k]).wait()

        def s_start(b, k):
            pltpu.async_copy(sbufs[k], acc.at[idx_c.at[b]], ssems[k], add=True)

        def s_wait(b, k):
            pltpu.make_async_copy(sbufs[k], acc.at[idx_c.at[b]],
                                  ssems[k]).wait()

        def scale(b, k):
            gb = gbufs[k]
            sb = sbufs[k]

            def sgrp(j, _):
                base = j * 16
                if has_w:
                    wvec = w_v[b, pl.ds(j * 16, 16)]
                for e in range(16):
                    r = base + e
                    for f in range(F // 16):
                        v = gb[r, pl.ds(f * 16, 16)]
                        if has_w:
                            v = v * wvec[e]
                        sb[r, pl.ds(f * 16, 16)] = v
                return 0

            lax.fori_loop(0, B // 16, sgrp, 0)

        for zi in range(RPT // 16):
            pltpu.sync_copy(zb, acc.at[pl.ds(s * RPT + zi * 16, 16)])
        plsc.subcore_barrier()

        def chunk_body(ch, _):
            base = start + s * pt + ch * CHUNK
            srow = jnp.minimum(base, EB - CHUNK)
            bo = base - srow
            nb = jnp.minimum(jnp.minimum(CHUNK, pt - ch * CHUNK), EB - base)

            pltpu.sync_copy(row_h.at[pl.ds(srow, CHUNK)], idx_r)
            pltpu.sync_copy(col_h.at[pl.ds(srow, CHUNK)], idx_c)
            if has_w:
                pltpu.sync_copy(w_h.at[pl.ds(srow, CHUNK)], w_v)

            for k in range(NBUF):
                g_start(bo + k, k)

            def mbody(g, _):
                for k in range(NBUF):
                    b = bo + g * NBUF + k
                    g_wait(b, k)

                    @pl.when(g > 0)
                    def _():
                        s_wait(b - NBUF, k)

                    scale(b, k)
                    s_start(b, k)

                    @pl.when(g * NBUF + k + NBUF < nb)
                    def _():
                        g_start(b + NBUF, k)
                return 0

            lax.fori_loop(0, nb // NBUF, mbody, 0)
            for k in range(NBUF):
                s_wait(bo + nb - NBUF + k, k)
            return 0

        lax.fori_loop(0, nchunks, chunk_body, 0)

        plsc.subcore_barrier()
        pltpu.sync_copy(acc.at[pl.ds(s * RPT, RPT)],
                        out_h.at[st, pl.ds(s * RPT, RPT)])
        plsc.subcore_barrier()

    @pl.when(c == 1)
    def _():
        process_set(0, tab_h.at[0], rowa_h, cola_h, None)
        process_set(3, tab_h.at[1], rowb_h, colb_h, wb_h,
                    start=16 * BMT, pt=BXT, nchunks=1)

    @pl.when(c == 0)
    def _():
        process_set(1, tab_h.at[1], rowb_h, colb_h, wb_h,
                    start=0, pt=BMT, nchunks=2)
        process_set(2, tab_h.at[2], rowc_h, colc_h, wc_h)


_gs = functools.partial(
    pl.kernel,
    out_type=jax.ShapeDtypeStruct((4, NP, F), jnp.float32),
    mesh=_mesh,
    compiler_params=pltpu.CompilerParams(needs_layout_passes=False, use_tc_tiling_on_sc=False),
    scratch_types=[
        pltpu.VMEM((CHUNK, B), jnp.int32),
        pltpu.VMEM((CHUNK, B), jnp.int32),
        pltpu.VMEM((CHUNK, B), jnp.float32),
        pltpu.VMEM_SHARED((NP, F), jnp.float32),
        pltpu.VMEM((16, F), jnp.float32),
    ]
    + [pltpu.VMEM((B, F), jnp.float32)] * (2 * NBUF)
    + [pltpu.SemaphoreType.DMA] * (2 * NBUF),
)(_gs_body)


def _tc0_body(xp_ref, w1_ref, degp_ref, hp_ref, dis_ref):
    deg = jnp.sum(degp_ref[...], axis=0) + 1.0
    dis = lax.rsqrt(deg)
    dis_ref[...] = dis
    h = jnp.dot(xp_ref[...], w1_ref[...], preferred_element_type=jnp.float32)
    hp_ref[...] = dis[:, :, None] * h[None, :, :]


_tc0 = pl.pallas_call(
    _tc0_body,
    grid=(NP // RB,),
    in_specs=[
        pl.BlockSpec((RB, D), lambda i: (i, 0)),
        pl.BlockSpec((D, F), lambda i: (0, 0)),
        pl.BlockSpec((NW, 3, RB), lambda i: (0, 0, i)),
    ],
    out_specs=[
        pl.BlockSpec((3, RB, F), lambda i: (0, i, 0)),
        pl.BlockSpec((3, RB), lambda i: (0, i)),
    ],
    out_shape=[
        jax.ShapeDtypeStruct((3, NP, F), jnp.float32),
        jax.ShapeDtypeStruct((3, NP), jnp.float32),
    ],
)


def _combine(acc_ref, hp_ref, dis_ref, b_ref):
    dis = dis_ref[...]
    a4 = acc_ref[...]
    acc = jnp.stack([a4[0], a4[1] + a4[3], a4[2]])
    h = dis[:, :, None] * (acc + hp_ref[...]) + b_ref[...]
    x = jnp.maximum(h, 0.0)
    return jnp.concatenate([x[0], x[1], x[2]], axis=-1), dis


def _tc1_body(acc_ref, hp_ref, dis_ref, w2_ref, b1_ref, out_ref):
    xcat, dis = _combine(acc_ref, hp_ref, dis_ref, b1_ref)
    h2 = jnp.dot(xcat, w2_ref[...], preferred_element_type=jnp.float32)
    out_ref[...] = dis[:, :, None] * h2[None, :, :]


_tc1 = pl.pallas_call(
    _tc1_body,
    grid=(NP // RB,),
    in_specs=[
        pl.BlockSpec((4, RB, F), lambda i: (0, i, 0)),
        pl.BlockSpec((3, RB, F), lambda i: (0, i, 0)),
        pl.BlockSpec((3, RB), lambda i: (0, i)),
        pl.BlockSpec((F3, F), lambda i: (0, 0)),
        pl.BlockSpec((1, F), lambda i: (0, 0)),
    ],
    out_specs=pl.BlockSpec((3, RB, F), lambda i: (0, i, 0)),
    out_shape=jax.ShapeDtypeStruct((3, NP, F), jnp.float32),
)


def _tc2_body(acc_ref, hp_ref, dis_ref, cw_ref, cb_ref, b2_ref, out_ref):
    xcat, _ = _combine(acc_ref, hp_ref, dis_ref, b2_ref)
    logits = jnp.dot(xcat, cw_ref[...], preferred_element_type=jnp.float32)
    logits = logits + cb_ref[...][None, :]
    m = jnp.max(logits, axis=1, keepdims=True)
    lse = jnp.log(jnp.sum(jnp.exp(logits - m), axis=1, keepdims=True)) + m
    out_ref[...] = logits - lse


_tc2 = pl.pallas_call(
    _tc2_body,
    grid=(NP // RB,),
    in_specs=[
        pl.BlockSpec((4, RB, F), lambda i: (0, i, 0)),
        pl.BlockSpec((3, RB, F), lambda i: (0, i, 0)),
        pl.BlockSpec((3, RB), lambda i: (0, i)),
        pl.BlockSpec((F3, C), lambda i: (0, 0)),
        pl.BlockSpec((C,), lambda i: (0,)),
        pl.BlockSpec((1, F), lambda i: (0, 0)),
    ],
    out_specs=pl.BlockSpec((RB, C), lambda i: (i, 0)),
    out_shape=jax.ShapeDtypeStruct((NP, C), jnp.float32),
)


def kernel(x, edge_index, edge_in, edge_out, in_w, out_w,
           lin1_w, lin2_w, conv_w, conv_b, bias1, bias2):
    xp = jnp.pad(x, ((0, NP - N), (0, 0)))
    ra = edge_index[0].reshape(EB, B)
    ca = edge_index[1].reshape(EB, B)
    rb = edge_in[0].reshape(EB, B)
    cb = edge_in[1].reshape(EB, B)
    rc = edge_out[0].reshape(EB, B)
    cc = edge_out[1].reshape(EB, B)
    wb = in_w.reshape(EB, B)
    wc = out_w.reshape(EB, B)

    degp = _deg(ca, cb, cc, wb, wc).reshape(NW, 3, NP)
    hp, dis = _tc0(xp, lin1_w, degp)
    acc1 = _gs(ra, rb, rc, ca, cb, cc, wb, wc, hp)
    hp2 = _tc1(acc1, hp, dis, lin2_w, bias1)
    acc2 = _gs(ra, rb, rc, ca, cb, cc, wb, wc, hp2)
    out = _tc2(acc2, hp2, dis, conv_w, conv_b, bias2)
    return out[:N]


# speedup vs baseline: 2.9633x; 1.0518x over previous
import functools

import jax
import jax.numpy as jnp
from jax import lax
from jax.experimental import pallas as pl
from jax.experimental.pallas import tpu as pltpu
from jax.experimental.pallas import tpu_sc as plsc

N, D, F, C, E = 10000, 128, 64, 64, 320000
NC, NS = 2, 16
NW = NC * NS
NP = 10240
B = 80
NBUF = 4
RPT = NP // NS
RB = 256
F3 = 3 * F

EB = E // B
CHUNK = 128
TPB = 2 * CHUNK
BMT = 120
BXT = 132

_mesh = plsc.VectorSubcoreMesh(core_axis_name="c", subcore_axis_name="s")


def _deg_body(cola_h, colb_h, colc_h, wb_h, wc_h, out_h, col_v, w_v, deg_v):
    c = lax.axis_index("c")
    s = lax.axis_index("s")
    wid = s * NC + c
    srow = wid * (EB // NW)
    nbd = EB // NW
    z = jnp.zeros((16,), jnp.float32)
    one = jnp.ones((16,), jnp.float32)

    def zbody(i, _):
        deg_v[pl.ds(i * 16, 16)] = z
        return 0

    lax.fori_loop(0, 3 * NP // 16, zbody, 0)

    for st, (col_h, w_h) in enumerate(
            ((cola_h, None), (colb_h, wb_h), (colc_h, wc_h))):
        pltpu.sync_copy(col_h.at[pl.ds(srow, EB // NW)], col_v)
        if w_h is not None:
            pltpu.sync_copy(w_h.at[pl.ds(srow, EB // NW)], w_v)

        def ebody(i, _, st=st, has_w=w_h is not None):
            r = i // (B // 16)
            j = i % (B // 16)
            idx = col_v[r, pl.ds(j * 16, 16)] + (st * NP)
            wv = w_v[r, pl.ds(j * 16, 16)] if has_w else one
            plsc.addupdate_scatter(deg_v, [idx], wv)
            return 0

        lax.fori_loop(0, nbd * (B // 16), ebody, 0)
    pltpu.sync_copy(deg_v, out_h.at[pl.ds(wid * 3 * NP, 3 * NP)])


_deg = functools.partial(
    pl.kernel,
    out_type=jax.ShapeDtypeStruct((NW * 3 * NP,), jnp.float32),
    mesh=_mesh,
    compiler_params=pltpu.CompilerParams(needs_layout_passes=False, use_tc_tiling_on_sc=False),
    scratch_types=[
        pltpu.VMEM((EB // NW, B), jnp.int32),
        pltpu.VMEM((EB // NW, B), jnp.float32),
        pltpu.VMEM((3 * NP,), jnp.float32),
    ],
)(_deg_body)


def _gs_body(rowa_h, rowb_h, rowc_h, cola_h, colb_h, colc_h, wb_h, wc_h,
             tab_h, out_h,
             idx_r, idx_c, w_v, acc, zb,
             g0, g1, g2, g3, s0, s1, s2, s3,
             gm0, gm1, gm2, gm3, sm0, sm1, sm2, sm3):
    c = lax.axis_index("c")
    s = lax.axis_index("s")
    gbufs = (g0, g1, g2, g3)
    sbufs = (s0, s1, s2, s3)
    gsems = (gm0, gm1, gm2, gm3)
    ssems = (sm0, sm1, sm2, sm3)

    z = jnp.zeros((16,), jnp.float32)

    def zb_body(i, _):
        for f in range(F // 16):
            zb[i, pl.ds(f * 16, 16)] = z
        return 0

    lax.fori_loop(0, 16, zb_body, 0)

    def process_set(st, tab_h, row_h, col_h, w_h, start=0, pt=TPB,
                    nchunks=2):
        has_w = w_h is not None

        def g_start(b, k):
            pltpu.async_copy(tab_h.at[idx_r.at[b]], gbufs[k], gsems[k])

        def g_wait(b, k):
            pltpu.make_async_copy(tab_h.at[idx_r.at[b]], gbufs[k],
                                  gsems[k]).wait()

        def s_start(b, k):
            pltpu.async_copy(sbufs[k], acc.at[idx_c.at[b]], ssems[k], add=True)

        def s_wait(b, k):
            pltpu.make_async_copy(sbufs[k], acc.at[idx_c.at[b]],
                                  ssems[k]).wait()

        def scale(b, k):
            gb = gbufs[k]
            sb = sbufs[k]

            def sgrp(j, _):
                base = j * 16
                if has_w:
                    wvec = w_v[b, pl.ds(j * 16, 16)]
                for e in range(16):
                    r = base + e
                    for f in range(F // 16):
                        v = gb[r, pl.ds(f * 16, 16)]
                        if has_w:
                            v = v * wvec[e]
                        sb[r, pl.ds(f * 16, 16)] = v
                return 0

            lax.fori_loop(0, B // 16, sgrp, 0)

        for zi in range(RPT // 16):
            pltpu.sync_copy(zb, acc.at[pl.ds(s * RPT + zi * 16, 16)])
        plsc.subcore_barrier()

        def chunk_body(ch, _):
            base = start + s * pt + ch * CHUNK
            srow = jnp.minimum(base, EB - CHUNK)
            bo = base - srow
            nb = jnp.minimum(jnp.minimum(CHUNK, pt - ch * CHUNK), EB - base)

            pltpu.sync_copy(row_h.at[pl.ds(srow, CHUNK)], idx_r)
            pltpu.sync_copy(col_h.at[pl.ds(srow, CHUNK)], idx_c)
            if has_w:
                pltpu.sync_copy(w_h.at[pl.ds(srow, CHUNK)], w_v)

            for k in range(NBUF):
                g_start(bo + k, k)

            def mbody(g, _):
                for k in range(NBUF):
                    b = bo + g * NBUF + k
                    g_wait(b, k)

                    @pl.when(g > 0)
                    def _():
                        s_wait(b - NBUF, k)

                    scale(b, k)
                    s_start(b, k)

                    @pl.when(g * NBUF + k + NBUF < nb)
                    def _():
                        g_start(b + NBUF, k)
                return 0

            lax.fori_loop(0, nb // NBUF, mbody, 0)
            for k in range(NBUF):
                s_wait(bo + nb - NBUF + k, k)
            return 0

        lax.fori_loop(0, nchunks, chunk_body, 0)

        plsc.subcore_barrier()
        pltpu.sync_copy(acc.at[pl.ds(s * RPT, RPT)],
                        out_h.at[st, pl.ds(s * RPT, RPT)])
        plsc.subcore_barrier()

    @pl.when(c == 1)
    def _():
        process_set(0, tab_h.at[0], rowa_h, cola_h, None)
        process_set(3, tab_h.at[1], rowb_h, colb_h, wb_h,
                    start=16 * BMT, pt=BXT, nchunks=1)

    @pl.when(c == 0)
    def _():
        process_set(1, tab_h.at[1], rowb_h, colb_h, wb_h,
                    start=0, pt=BMT, nchunks=1)
        process_set(2, tab_h.at[2], rowc_h, colc_h, wc_h)


_gs = functools.partial(
    pl.kernel,
    out_type=jax.ShapeDtypeStruct((4, NP, F), jnp.float32),
    mesh=_mesh,
    compiler_params=pltpu.CompilerParams(needs_layout_passes=False, use_tc_tiling_on_sc=False),
    scratch_types=[
        pltpu.VMEM((CHUNK, B), jnp.int32),
        pltpu.VMEM((CHUNK, B), jnp.int32),
        pltpu.VMEM((CHUNK, B), jnp.float32),
        pltpu.VMEM_SHARED((NP, F), jnp.float32),
        pltpu.VMEM((16, F), jnp.float32),
    ]
    + [pltpu.VMEM((B, F), jnp.float32)] * (2 * NBUF)
    + [pltpu.SemaphoreType.DMA] * (2 * NBUF),
)(_gs_body)


def _tc0_body(xp_ref, w1_ref, degp_ref, hp_ref, dis_ref):
    deg = jnp.sum(degp_ref[...], axis=0) + 1.0
    dis = lax.rsqrt(deg)
    dis_ref[...] = dis
    h = jnp.dot(xp_ref[...], w1_ref[...], preferred_element_type=jnp.float32)
    hp_ref[...] = dis[:, :, None] * h[None, :, :]


_tc0 = pl.pallas_call(
    _tc0_body,
    grid=(NP // RB,),
    in_specs=[
        pl.BlockSpec((RB, D), lambda i: (i, 0)),
        pl.BlockSpec((D, F), lambda i: (0, 0)),
        pl.BlockSpec((NW, 3, RB), lambda i: (0, 0, i)),
    ],
    out_specs=[
        pl.BlockSpec((3, RB, F), lambda i: (0, i, 0)),
        pl.BlockSpec((3, RB), lambda i: (0, i)),
    ],
    out_shape=[
        jax.ShapeDtypeStruct((3, NP, F), jnp.float32),
        jax.ShapeDtypeStruct((3, NP), jnp.float32),
    ],
)


def _combine(acc_ref, hp_ref, dis_ref, b_ref):
    dis = dis_ref[...]
    a4 = acc_ref[...]
    acc = jnp.stack([a4[0], a4[1] + a4[3], a4[2]])
    h = dis[:, :, None] * (acc + hp_ref[...]) + b_ref[...]
    x = jnp.maximum(h, 0.0)
    return jnp.concatenate([x[0], x[1], x[2]], axis=-1), dis


def _tc1_body(acc_ref, hp_ref, dis_ref, w2_ref, b1_ref, out_ref):
    xcat, dis = _combine(acc_ref, hp_ref, dis_ref, b1_ref)
    h2 = jnp.dot(xcat, w2_ref[...], preferred_element_type=jnp.float32)
    out_ref[...] = dis[:, :, None] * h2[None, :, :]


_tc1 = pl.pallas_call(
    _tc1_body,
    grid=(NP // RB,),
    in_specs=[
        pl.BlockSpec((4, RB, F), lambda i: (0, i, 0)),
        pl.BlockSpec((3, RB, F), lambda i: (0, i, 0)),
        pl.BlockSpec((3, RB), lambda i: (0, i)),
        pl.BlockSpec((F3, F), lambda i: (0, 0)),
        pl.BlockSpec((1, F), lambda i: (0, 0)),
    ],
    out_specs=pl.BlockSpec((3, RB, F), lambda i: (0, i, 0)),
    out_shape=jax.ShapeDtypeStruct((3, NP, F), jnp.float32),
)


def _tc2_body(acc_ref, hp_ref, dis_ref, cw_ref, cb_ref, b2_ref, out_ref):
    xcat, _ = _combine(acc_ref, hp_ref, dis_ref, b2_ref)
    logits = jnp.dot(xcat, cw_ref[...], preferred_element_type=jnp.float32)
    logits = logits + cb_ref[...][None, :]
    m = jnp.max(logits, axis=1, keepdims=True)
    lse = jnp.log(jnp.sum(jnp.exp(logits - m), axis=1, keepdims=True)) + m
    out_ref[...] = logits - lse


_tc2 = pl.pallas_call(
    _tc2_body,
    grid=(NP // RB,),
    in_specs=[
        pl.BlockSpec((4, RB, F), lambda i: (0, i, 0)),
        pl.BlockSpec((3, RB, F), lambda i: (0, i, 0)),
        pl.BlockSpec((3, RB), lambda i: (0, i)),
        pl.BlockSpec((F3, C), lambda i: (0, 0)),
        pl.BlockSpec((C,), lambda i: (0,)),
        pl.BlockSpec((1, F), lambda i: (0, 0)),
    ],
    out_specs=pl.BlockSpec((RB, C), lambda i: (i, 0)),
    out_shape=jax.ShapeDtypeStruct((NP, C), jnp.float32),
)


def kernel(x, edge_index, edge_in, edge_out, in_w, out_w,
           lin1_w, lin2_w, conv_w, conv_b, bias1, bias2):
    xp = jnp.pad(x, ((0, NP - N), (0, 0)))
    ra = edge_index[0].reshape(EB, B)
    ca = edge_index[1].reshape(EB, B)
    rb = edge_in[0].reshape(EB, B)
    cb = edge_in[1].reshape(EB, B)
    rc = edge_out[0].reshape(EB, B)
    cc = edge_out[1].reshape(EB, B)
    wb = in_w.reshape(EB, B)
    wc = out_w.reshape(EB, B)

    degp = _deg(ca, cb, cc, wb, wc).reshape(NW, 3, NP)
    hp, dis = _tc0(xp, lin1_w, degp)
    acc1 = _gs(ra, rb, rc, ca, cb, cc, wb, wc, hp)
    hp2 = _tc1(acc1, hp, dis, lin2_w, bias1)
    acc2 = _gs(ra, rb, rc, ca, cb, cc, wb, wc, hp2)
    out = _tc2(acc2, hp2, dis, conv_w, conv_b, bias2)
    return out[:N]
